# baseline (speedup 1.0000x reference)
import functools

import jax
import jax.numpy as jnp
from jax import lax
from jax.experimental import pallas as pl
from jax.experimental.pallas import tpu as pltpu

_PACK = 16
_TILE = 1024
_DEPTH = 4
_NCORES = 2


def _round_up(n, m):
    return (n + m - 1) // m * m


def _ring_kernel(xp, w_ref, b_ref, op, xb, ob, in_sems, out_sems,
                 *, n_tiles):
    core = pl.program_id(0)
    row0 = core * (n_tiles * _TILE)

    def in_copy(slot, t):
        return pltpu.make_async_copy(
            xp.at[pl.ds(row0 + t * _TILE, _TILE), :],
            xb.at[slot],
            in_sems.at[slot],
        )

    def out_copy(slot, t):
        return pltpu.make_async_copy(
            ob.at[slot],
            op.at[pl.ds(row0 + t * _TILE, _TILE), :],
            out_sems.at[slot],
        )

    for s in range(min(_DEPTH, n_tiles)):
        in_copy(s, s).start()

    for t in range(n_tiles):
        slot = t % _DEPTH
        in_copy(slot, t).wait()
        if t >= _DEPTH:
            out_copy(slot, t - _DEPTH).wait()
        acc = lax.dot_general(
            xb[slot],
            w_ref[...],
            dimension_numbers=(((1,), (0,)), ((), ())),
            preferred_element_type=jnp.float32,
        )
        ob[slot] = acc + b_ref[...]
        out_copy(slot, t).start()
        if t + _DEPTH < n_tiles:
            in_copy(slot, t + _DEPTH).start()

    for t in range(max(0, n_tiles - _DEPTH), n_tiles):
        out_copy(t % _DEPTH, t).wait()


@jax.jit
def _forward(x, weight, bias2d):
    B, d_in = x.shape
    d_out = weight.shape[0]

    span = _NCORES * _TILE * _PACK
    B_p = _round_up(B, span)
    if B_p != B:
        x = jnp.pad(x, ((0, B_p - B), (0, 0)))
    n_tiles = B_p // span
    rows = B_p // _PACK

    xp = x.reshape(rows, _PACK * d_in)
    w_big = jnp.kron(jnp.eye(_PACK, dtype=weight.dtype), weight.T)
    b_big = jnp.tile(bias2d, (1, _PACK))

    kern = functools.partial(_ring_kernel, n_tiles=n_tiles)

    out = pl.pallas_call(
        kern,
        grid=(_NCORES,),
        in_specs=[
            pl.BlockSpec(memory_space=pl.ANY),
            pl.BlockSpec(memory_space=pltpu.MemorySpace.VMEM),
            pl.BlockSpec(memory_space=pltpu.MemorySpace.VMEM),
        ],
        out_specs=pl.BlockSpec(memory_space=pl.ANY),
        out_shape=jax.ShapeDtypeStruct((rows, _PACK * d_out), x.dtype),
        scratch_shapes=[
            pltpu.VMEM((_DEPTH, _TILE, _PACK * d_in), jnp.float32),
            pltpu.VMEM((_DEPTH, _TILE, _PACK * d_out), jnp.float32),
            pltpu.SemaphoreType.DMA((_DEPTH,)),
            pltpu.SemaphoreType.DMA((_DEPTH,)),
        ],
        compiler_params=pltpu.CompilerParams(
            dimension_semantics=("parallel",),
        ),
        cost_estimate=pl.CostEstimate(
            flops=2 * B_p * _PACK * d_in * d_out,
            bytes_accessed=B_p * (d_in + d_out) * 4,
            transcendentals=0,
        ),
    )(xp, w_big, b_big)

    out = out.reshape(B_p, d_out)
    if B_p != B:
        out = out[:B]
    return out


def kernel(x, weight, bias2d):
    return _forward(x, weight, bias2d)


# speedup vs baseline: 9.5020x; 9.5020x over previous
import jax
import jax.numpy as jnp
from jax import lax
from jax.experimental import pallas as pl
from jax.experimental.pallas import tpu as pltpu

_CHUNK = 8192


def _round_up(n, m):
    return (n + m - 1) // m * m


def _tmm_kernel(xt_ref, w_ref, bt_ref, ot_ref):
    acc = lax.dot_general(
        w_ref[...],
        xt_ref[...],
        dimension_numbers=(((1,), (0,)), ((), ())),
        preferred_element_type=jnp.float32,
    )
    ot_ref[...] = (acc + bt_ref[...]).astype(ot_ref.dtype)


@jax.jit
def _forward(x, weight, bias2d):
    B, d_in = x.shape
    d_out = weight.shape[0]

    B_p = _round_up(B, 2 * _CHUNK)
    if B_p != B:
        x = jnp.pad(x, ((0, B_p - B), (0, 0)))

    xt = x.T
    bt = bias2d.T

    out_t = pl.pallas_call(
        _tmm_kernel,
        grid=(B_p // _CHUNK,),
        in_specs=[
            pl.BlockSpec((d_in, _CHUNK), lambda i: (0, i)),
            pl.BlockSpec((d_out, d_in), lambda i: (0, 0)),
            pl.BlockSpec((d_out, 1), lambda i: (0, 0)),
        ],
        out_specs=pl.BlockSpec((d_out, _CHUNK), lambda i: (0, i)),
        out_shape=jax.ShapeDtypeStruct((d_out, B_p), x.dtype),
        compiler_params=pltpu.CompilerParams(
            dimension_semantics=("parallel",),
        ),
        cost_estimate=pl.CostEstimate(
            flops=2 * B_p * d_in * d_out,
            bytes_accessed=B_p * (d_in + d_out) * 4,
            transcendentals=0,
        ),
    )(xt, weight, bt)

    out = out_t.T
    if B_p != B:
        out = out[:B]
    return out


def kernel(x, weight, bias2d):
    return _forward(x, weight, bias2d)


# speedup vs baseline: 15.5579x; 1.6373x over previous
import jax
import jax.numpy as jnp
from jax import lax
from jax.experimental import pallas as pl
from jax.experimental.pallas import tpu as pltpu

_CHUNK = 32768


def _round_up(n, m):
    return (n + m - 1) // m * m


def _tmm_kernel(xt_ref, w_ref, bt_ref, ot_ref):
    acc = lax.dot_general(
        w_ref[...],
        xt_ref[...],
        dimension_numbers=(((1,), (0,)), ((), ())),
        preferred_element_type=jnp.float32,
    )
    ot_ref[...] = (acc + bt_ref[...]).astype(ot_ref.dtype)


@jax.jit
def _forward(x, weight, bias2d):
    B, d_in = x.shape
    d_out = weight.shape[0]

    B_p = _round_up(B, 2 * _CHUNK)
    if B_p != B:
        x = jnp.pad(x, ((0, B_p - B), (0, 0)))

    xt = x.T
    bt = bias2d.T

    out_t = pl.pallas_call(
        _tmm_kernel,
        grid=(B_p // _CHUNK,),
        in_specs=[
            pl.BlockSpec((d_in, _CHUNK), lambda i: (0, i)),
            pl.BlockSpec((d_out, d_in), lambda i: (0, 0)),
            pl.BlockSpec((d_out, 1), lambda i: (0, 0)),
        ],
        out_specs=pl.BlockSpec((d_out, _CHUNK), lambda i: (0, i)),
        out_shape=jax.ShapeDtypeStruct((d_out, B_p), x.dtype),
        compiler_params=pltpu.CompilerParams(
            dimension_semantics=("parallel",),
        ),
        cost_estimate=pl.CostEstimate(
            flops=2 * B_p * d_in * d_out,
            bytes_accessed=B_p * (d_in + d_out) * 4,
            transcendentals=0,
        ),
    )(xt, weight, bt)

    out = out_t.T
    if B_p != B:
        out = out[:B]
    return out


def kernel(x, weight, bias2d):
    return _forward(x, weight, bias2d)


# speedup vs baseline: 16.3820x; 1.0530x over previous
import jax
import jax.numpy as jnp
from jax import lax
from jax.experimental import pallas as pl
from jax.experimental.pallas import tpu as pltpu

_CHUNK = 65536


def _round_up(n, m):
    return (n + m - 1) // m * m


def _tmm_kernel(xt_ref, w_ref, bt_ref, ot_ref):
    acc = lax.dot_general(
        w_ref[...],
        xt_ref[...],
        dimension_numbers=(((1,), (0,)), ((), ())),
        preferred_element_type=jnp.float32,
    )
    ot_ref[...] = (acc + bt_ref[...]).astype(ot_ref.dtype)


@jax.jit
def _forward(x, weight, bias2d):
    B, d_in = x.shape
    d_out = weight.shape[0]

    B_p = _round_up(B, 2 * _CHUNK)
    if B_p != B:
        x = jnp.pad(x, ((0, B_p - B), (0, 0)))

    xt = x.T
    bt = bias2d.T

    out_t = pl.pallas_call(
        _tmm_kernel,
        grid=(B_p // _CHUNK,),
        in_specs=[
            pl.BlockSpec((d_in, _CHUNK), lambda i: (0, i)),
            pl.BlockSpec((d_out, d_in), lambda i: (0, 0)),
            pl.BlockSpec((d_out, 1), lambda i: (0, 0)),
        ],
        out_specs=pl.BlockSpec((d_out, _CHUNK), lambda i: (0, i)),
        out_shape=jax.ShapeDtypeStruct((d_out, B_p), x.dtype),
        compiler_params=pltpu.CompilerParams(
            dimension_semantics=("parallel",),
        ),
        cost_estimate=pl.CostEstimate(
            flops=2 * B_p * d_in * d_out,
            bytes_accessed=B_p * (d_in + d_out) * 4,
            transcendentals=0,
        ),
    )(xt, weight, bt)

    out = out_t.T
    if B_p != B:
        out = out[:B]
    return out


def kernel(x, weight, bias2d):
    return _forward(x, weight, bias2d)
